# baseline (speedup 1.0000x reference)
import functools

import jax
import jax.numpy as jnp
from jax import lax
from jax.experimental import pallas as pl
from jax.experimental.pallas import tpu as pltpu
from jax.experimental.pallas import tpu_sc as plsc

B = 128
V = 100000
L = 16
HALFV = V // 2
CH = 192
NFULL = 260
TAILW = HALFV - NFULL * CH
NEG_INF = float("-inf")


def _build_sc_kernel():
  mesh = plsc.VectorSubcoreMesh(core_axis_name="c", subcore_axis_name="s")

  @functools.partial(
      pl.kernel,
      mesh=mesh,
      compiler_params=pltpu.CompilerParams(needs_layout_passes=False,
                                           skip_device_barrier=True,
                                           use_tc_tiling_on_sc=True),
      out_type=jax.ShapeDtypeStruct((16, B), jnp.float32),
      scratch_types=[
          pltpu.VMEM((B,), jnp.int32),
          pltpu.VMEM((8, B), jnp.float32),
          pltpu.VMEM((B,), jnp.float32),
          pltpu.VMEM((CH, B), jnp.float32),
          pltpu.VMEM((CH, B), jnp.float32),
          pltpu.VMEM((CH, B), jnp.float32),
          pltpu.VMEM((L,), jnp.float32),
          pltpu.VMEM((8, B), jnp.float32),
          pltpu.VMEM((16 * B,), jnp.float32),
          pltpu.VMEM_SHARED((B,), jnp.float32),
          pltpu.VMEM_SHARED((16 * B,), jnp.float32),
          pltpu.SemaphoreType.DMA,
          pltpu.SemaphoreType.DMA,
          pltpu.SemaphoreType.DMA,
      ],
  )
  def sc_kernel(xt_hbm, targets_hbm, out_hbm,
                tgt_v, vblk, vall, cbuf0, cbuf1, cbuf2, stage_v, rank_st,
                red_v, shared_v, shared_r, sem0, sem1, sem2):
    c = lax.axis_index("c")
    s = lax.axis_index("s")
    iota = lax.iota(jnp.int32, L)

    pltpu.sync_copy(targets_hbm, tgt_v)

    vmine = jnp.zeros((L,), jnp.float32)
    r0 = s * 8
    tb0 = jnp.minimum(r0, B - L)
    traw = tgt_v[pl.ds(tb0, L)]
    for j in range(8):
      t_j = jnp.max(jnp.where(iota == (r0 - tb0) + j, traw, -1))
      blk0 = (t_j // 8) * 8
      pltpu.sync_copy(xt_hbm.at[pl.ds(blk0, 8), :], vblk)
      b_abs = r0 + j
      bseg = (b_abs // L) * L
      vvec = vblk[t_j - blk0, pl.ds(bseg, L)]
      v_j = jnp.max(jnp.where(iota == b_abs - bseg, vvec, NEG_INF))
      vmine = vmine + jnp.where(iota == j, v_j, jnp.float32(0))
    stage_v[...] = vmine
    pltpu.sync_copy(stage_v.at[pl.ds(0, 8)], shared_v.at[pl.ds(r0, 8)])
    plsc.subcore_barrier()
    pltpu.sync_copy(shared_v, vall)

    tvs, vvs, vhis = [], [], []
    for g in range(8):
      tv = tgt_v[pl.ds(g * L, L)]
      vv = vall[pl.ds(g * L, L)]
      bits = plsc.bitcast(vv, jnp.int32)
      hi_bits = jnp.where(
          bits == jnp.int32(-2147483648),
          jnp.int32(1),
          jnp.where(bits < 0, bits - 1, bits + 1))
      vhi = plsc.bitcast(hi_bits, jnp.float32)
      tvs.append(tv)
      vvs.append(vv)
      vhis.append(vhi)

    def process(buf, lo, ch, accs):
      hi = lo + ch
      inb = jnp.zeros((L,), jnp.int32)
      for g in range(8):
        inb = inb + jnp.logical_and(tvs[g] >= lo, tvs[g] < hi).astype(jnp.int32)
      has_b = jnp.max(inb) > 0
      wvs = [jnp.where(tvs[g] >= hi, vvs[g], vhis[g]) for g in range(8)]

      def fast(aa):
        def body(jj, a, buf=buf):
          return tuple(
              a[g] + (buf[jj, pl.ds(g * L, L)] >= wvs[g]).astype(jnp.int32)
              for g in range(8))
        return plsc.parallel_loop(0, ch, unroll=4, carry=aa)(body)

      def slow(aa):
        def body(jj, a, buf=buf, lo=lo):
          j_vec = jnp.full((L,), lo + jj, jnp.int32)
          out = []
          for g in range(8):
            x = buf[jj, pl.ds(g * L, L)]
            wv = jnp.where(j_vec < tvs[g], vvs[g], vhis[g])
            m = jnp.logical_and(x >= wv, tvs[g] != j_vec)
            out.append(a[g] + m.astype(jnp.int32))
          return tuple(out)
        return plsc.parallel_loop(0, ch, unroll=2, carry=aa)(body)

      return list(lax.cond(has_b, slow, fast, tuple(accs)))

    half0 = c * HALFV
    bufs = (cbuf0, cbuf1, cbuf2)
    sems = (sem0, sem1, sem2)
    NBUF = 3

    def chunk_off(k):
      return half0 + (s + 16 * k) * CH

    def fire(k):
      return pltpu.async_copy(
          xt_hbm.at[pl.ds(chunk_off(k), CH), :],
          bufs[k % NBUF], sems[k % NBUF])

    accs = [jnp.zeros((L,), jnp.int32) for _ in range(8)]
    cps = [fire(k) for k in range(NBUF)]
    for k in range(16):
      cps[k % NBUF].wait()
      accs = process(bufs[k % NBUF], chunk_off(k), CH, accs)
      if k + NBUF < 16:
        cps[k % NBUF] = fire(k + NBUF)

    extra_off = half0 + jnp.minimum(256 + s, NFULL - 1) * CH
    tail_off = half0 + NFULL * CH

    @pl.when(s < 4)
    def _():
      pltpu.async_copy(xt_hbm.at[pl.ds(extra_off, CH), :], cbuf0, sem0).wait()

    @pl.when(s == 4)
    def _():
      pltpu.async_copy(xt_hbm.at[pl.ds(tail_off, TAILW), :],
                       cbuf0.at[pl.ds(0, TAILW), :], sem0).wait()

    zero8 = [jnp.zeros((L,), jnp.int32) for _ in range(8)]
    accs2 = process(cbuf0, extra_off, CH, zero8)
    accs3 = process(cbuf0, tail_off, TAILW, list(zero8))
    m_extra = jnp.full((L,), s < 4)
    m_tail = jnp.full((L,), s == 4)
    for g in range(8):
      accs[g] = (accs[g]
                 + jnp.where(m_extra, accs2[g], 0)
                 + jnp.where(m_tail, accs3[g], 0))

    for g in range(8):
      stage_v[...] = accs[g].astype(jnp.float32)
      pltpu.sync_copy(stage_v, shared_r.at[pl.ds(s * B + g * L, L)])
    plsc.subcore_barrier()

    @pl.when(s == 0)
    def _():
      pltpu.sync_copy(shared_r, red_v)
      tot = [jnp.zeros((L,), jnp.float32) for _ in range(8)]
      for i in range(16):
        for g in range(8):
          tot[g] = tot[g] + red_v[pl.ds(i * B + g * L, L)]
      for g in range(8):
        rank_st[0, pl.ds(g * L, L)] = tot[g]
      pltpu.sync_copy(rank_st, out_hbm.at[pl.ds(c * 8, 8), :])

  return sc_kernel


def _tc_count(r_ref, o_ref):
  ranks = r_ref[0:1, :] + r_ref[8:9, :]
  c1 = jnp.sum((ranks < 1.0).astype(jnp.float32))
  c5 = jnp.sum((ranks < 5.0).astype(jnp.float32))
  c10 = jnp.sum((ranks < 10.0).astype(jnp.float32))
  rowi = lax.broadcasted_iota(jnp.int32, (8, B), 0)
  coli = lax.broadcasted_iota(jnp.int32, (8, B), 1)
  first = rowi == 0
  o_ref[...] = (jnp.where(jnp.logical_and(first, coli == 0), c1, 0.0)
                + jnp.where(jnp.logical_and(first, coli == 1), c5, 0.0)
                + jnp.where(jnp.logical_and(first, coli == 2), c10, 0.0))


@jax.jit
def kernel(outputs, targets):
  sc = _build_sc_kernel()
  partials = sc(outputs.T, targets.astype(jnp.int32))
  counts = pl.pallas_call(
      _tc_count,
      out_shape=jax.ShapeDtypeStruct((8, B), jnp.float32),
  )(partials)
  return (counts[0, 0], counts[0, 1], counts[0, 2])


# speedup vs baseline: 1.0426x; 1.0426x over previous
import functools

import jax
import jax.numpy as jnp
from jax import lax
from jax.experimental import pallas as pl
from jax.experimental.pallas import tpu as pltpu
from jax.experimental.pallas import tpu_sc as plsc

B = 128
V = 100000
L = 16
HALFV = V // 2
CH = 192
NFULL = 260
TAILW = HALFV - NFULL * CH
NEG_INF = float("-inf")


def _build_sc_kernel():
  mesh = plsc.VectorSubcoreMesh(core_axis_name="c", subcore_axis_name="s")

  @functools.partial(
      pl.kernel,
      mesh=mesh,
      compiler_params=pltpu.CompilerParams(needs_layout_passes=False,
                                           skip_device_barrier=True,
                                           use_tc_tiling_on_sc=True),
      out_type=jax.ShapeDtypeStruct((16, B), jnp.float32),
      scratch_types=[
          pltpu.VMEM((B,), jnp.int32),
          pltpu.VMEM((64, B), jnp.float32),
          pltpu.VMEM((B,), jnp.float32),
          pltpu.VMEM((CH, B), jnp.float32),
          pltpu.VMEM((CH, B), jnp.float32),
          pltpu.VMEM((CH, B), jnp.float32),
          pltpu.VMEM((L,), jnp.float32),
          pltpu.VMEM((8, B), jnp.float32),
          pltpu.VMEM((16 * B,), jnp.float32),
          pltpu.VMEM_SHARED((B,), jnp.float32),
          pltpu.VMEM_SHARED((16 * B,), jnp.float32),
          pltpu.SemaphoreType.DMA,
          pltpu.SemaphoreType.DMA,
          pltpu.SemaphoreType.DMA,
      ],
  )
  def sc_kernel(xt_hbm, targets_hbm, out_hbm,
                tgt_v, vblk, vall, cbuf0, cbuf1, cbuf2, stage_v, rank_st,
                red_v, shared_v, shared_r, sem0, sem1, sem2):
    c = lax.axis_index("c")
    s = lax.axis_index("s")
    iota = lax.iota(jnp.int32, L)

    pltpu.sync_copy(targets_hbm, tgt_v)

    vmine = jnp.zeros((L,), jnp.float32)
    r0 = s * 8
    tb0 = jnp.minimum(r0, B - L)
    traw = tgt_v[pl.ds(tb0, L)]
    tjs = []
    cps_v = []
    for j in range(8):
      t_j = jnp.max(jnp.where(iota == (r0 - tb0) + j, traw, -1))
      blk0 = (t_j // 8) * 8
      cps_v.append(pltpu.async_copy(
          xt_hbm.at[pl.ds(blk0, 8), :], vblk.at[pl.ds(j * 8, 8), :], sem0))
      tjs.append(t_j)
    for j in range(8):
      cps_v[j].wait()
    for j in range(8):
      t_j = tjs[j]
      blk0 = (t_j // 8) * 8
      b_abs = r0 + j
      bseg = (b_abs // L) * L
      vvec = vblk[j * 8 + (t_j - blk0), pl.ds(bseg, L)]
      v_j = jnp.max(jnp.where(iota == b_abs - bseg, vvec, NEG_INF))
      vmine = vmine + jnp.where(iota == j, v_j, jnp.float32(0))
    stage_v[...] = vmine
    pltpu.sync_copy(stage_v.at[pl.ds(0, 8)], shared_v.at[pl.ds(r0, 8)])
    plsc.subcore_barrier()
    pltpu.sync_copy(shared_v, vall)

    tvs, vvs, vhis = [], [], []
    for g in range(8):
      tv = tgt_v[pl.ds(g * L, L)]
      vv = vall[pl.ds(g * L, L)]
      bits = plsc.bitcast(vv, jnp.int32)
      hi_bits = jnp.where(
          bits == jnp.int32(-2147483648),
          jnp.int32(1),
          jnp.where(bits < 0, bits - 1, bits + 1))
      vhi = plsc.bitcast(hi_bits, jnp.float32)
      tvs.append(tv)
      vvs.append(vv)
      vhis.append(vhi)

    def process(buf, lo, ch, accs):
      hi = lo + ch
      inb = jnp.zeros((L,), jnp.int32)
      for g in range(8):
        inb = inb + jnp.logical_and(tvs[g] >= lo, tvs[g] < hi).astype(jnp.int32)
      has_b = jnp.max(inb) > 0
      wvs = [jnp.where(tvs[g] >= hi, vvs[g], vhis[g]) for g in range(8)]

      def fast(aa):
        def body(jj, a, buf=buf):
          return tuple(
              a[g] + (buf[jj, pl.ds(g * L, L)] >= wvs[g]).astype(jnp.int32)
              for g in range(8))
        return plsc.parallel_loop(0, ch, unroll=4, carry=aa)(body)

      def slow(aa):
        def body(jj, a, buf=buf, lo=lo):
          j_vec = jnp.full((L,), lo + jj, jnp.int32)
          out = []
          for g in range(8):
            x = buf[jj, pl.ds(g * L, L)]
            wv = jnp.where(j_vec < tvs[g], vvs[g], vhis[g])
            m = jnp.logical_and(x >= wv, tvs[g] != j_vec)
            out.append(a[g] + m.astype(jnp.int32))
          return tuple(out)
        return plsc.parallel_loop(0, ch, unroll=2, carry=aa)(body)

      return list(lax.cond(has_b, slow, fast, tuple(accs)))

    half0 = c * HALFV
    bufs = (cbuf0, cbuf1, cbuf2)
    sems = (sem0, sem1, sem2)
    NBUF = 3

    def chunk_off(k):
      return half0 + (s + 16 * k) * CH

    def fire(k):
      return pltpu.async_copy(
          xt_hbm.at[pl.ds(chunk_off(k), CH), :],
          bufs[k % NBUF], sems[k % NBUF])

    accs = [jnp.zeros((L,), jnp.int32) for _ in range(8)]
    cps = [fire(k) for k in range(NBUF)]
    for k in range(16):
      cps[k % NBUF].wait()
      accs = process(bufs[k % NBUF], chunk_off(k), CH, accs)
      if k + NBUF < 16:
        cps[k % NBUF] = fire(k + NBUF)

    extra_off = half0 + jnp.minimum(256 + s, NFULL - 1) * CH
    tail_off = half0 + NFULL * CH

    @pl.when(s < 4)
    def _():
      pltpu.async_copy(xt_hbm.at[pl.ds(extra_off, CH), :], cbuf0, sem0).wait()

    @pl.when(s == 4)
    def _():
      pltpu.async_copy(xt_hbm.at[pl.ds(tail_off, TAILW), :],
                       cbuf0.at[pl.ds(0, TAILW), :], sem0).wait()

    zero8 = [jnp.zeros((L,), jnp.int32) for _ in range(8)]
    accs2 = process(cbuf0, extra_off, CH, zero8)
    accs3 = process(cbuf0, tail_off, TAILW, list(zero8))
    m_extra = jnp.full((L,), s < 4)
    m_tail = jnp.full((L,), s == 4)
    for g in range(8):
      accs[g] = (accs[g]
                 + jnp.where(m_extra, accs2[g], 0)
                 + jnp.where(m_tail, accs3[g], 0))

    for g in range(8):
      stage_v[...] = accs[g].astype(jnp.float32)
      pltpu.sync_copy(stage_v, shared_r.at[pl.ds(s * B + g * L, L)])
    plsc.subcore_barrier()

    @pl.when(s == 0)
    def _():
      pltpu.sync_copy(shared_r, red_v)
      tot = [jnp.zeros((L,), jnp.float32) for _ in range(8)]
      for i in range(16):
        for g in range(8):
          tot[g] = tot[g] + red_v[pl.ds(i * B + g * L, L)]
      for g in range(8):
        rank_st[0, pl.ds(g * L, L)] = tot[g]
      pltpu.sync_copy(rank_st, out_hbm.at[pl.ds(c * 8, 8), :])

  return sc_kernel


def _tc_count(r_ref, o_ref):
  ranks = r_ref[0:1, :] + r_ref[8:9, :]
  c1 = jnp.sum((ranks < 1.0).astype(jnp.float32))
  c5 = jnp.sum((ranks < 5.0).astype(jnp.float32))
  c10 = jnp.sum((ranks < 10.0).astype(jnp.float32))
  rowi = lax.broadcasted_iota(jnp.int32, (8, B), 0)
  coli = lax.broadcasted_iota(jnp.int32, (8, B), 1)
  first = rowi == 0
  o_ref[...] = (jnp.where(jnp.logical_and(first, coli == 0), c1, 0.0)
                + jnp.where(jnp.logical_and(first, coli == 1), c5, 0.0)
                + jnp.where(jnp.logical_and(first, coli == 2), c10, 0.0))


@jax.jit
def kernel(outputs, targets):
  sc = _build_sc_kernel()
  partials = sc(outputs.T, targets.astype(jnp.int32))
  counts = pl.pallas_call(
      _tc_count,
      out_shape=jax.ShapeDtypeStruct((8, B), jnp.float32),
  )(partials)
  return (counts[0, 0], counts[0, 1], counts[0, 2])
